# Initial kernel scaffold; baseline (speedup 1.0000x reference)
import jax
import jax.numpy as jnp
from jax.experimental import pallas as pl


def kernel(atom_list, bond_list, atom_degree_list, bond_degree_list, atom_mask, params):
    raise NotImplementedError("write your pallas kernel here")


# speedup vs baseline: 9.1104x; 9.1104x over previous
import jax
import jax.numpy as jnp
from jax.experimental import pallas as pl

FP = 64
B = 256
L = 256
K = 6
AF = 39

_NEG = -9e8


def _lk(x):
    return jax.nn.leaky_relu(x, 0.2)


def _fp_kernel(x_ref, idx_ref, am_ref,
               wa_ref, ba_ref,
               alwa_ref, alwb_ref, alb_ref,
               atw_ref, atb_ref,
               gk_ref, grk_ref, gbi_ref, gbr_ref,
               mola_ref, molbw_ref, molb_ref,
               maw_ref, mab_ref,
               mgk_ref, mgrk_ref, mgbi_ref, mgbr_ref,
               l1w_ref, l1b_ref, l2w_ref, l2b_ref,
               ow_ref, ob_ref,
               out_ref):
    f32 = jnp.float32
    x = x_ref[0]
    idx = idx_ref[0]
    am = am_ref[0]

    af = _lk(jnp.dot(x, wa_ref[...], preferred_element_type=f32) + ba_ref[...])
    act = _lk(af)

    g = jnp.dot(act, alwb_ref[...], preferred_element_type=f32)
    a_sc = jnp.dot(af, alwa_ref[...], preferred_element_type=f32)
    iota = jax.lax.broadcasted_iota(jnp.int32, (L, L), 1)
    ohs = []
    gvs = []
    for k in range(K):
        oh = (idx[:, k:k + 1] == iota).astype(f32)
        ohs.append(oh)
        gvs.append(jnp.dot(oh, g, preferred_element_type=f32))
    gv = jnp.concatenate(gvs, axis=1)
    pad = idx == (L - 1)
    sc = _lk(a_sc + gv + alb_ref[...]) + jnp.where(pad, _NEG, 0.0)
    mx = jnp.max(sc, axis=1, keepdims=True)
    e = jnp.exp(sc - mx)
    aw = e / jnp.sum(e, axis=1, keepdims=True)
    aw = aw * jnp.where(pad, 0.0, 1.0)

    S = aw[:, 0:1] * ohs[0]
    for k in range(1, K):
        S = S + aw[:, k:k + 1] * ohs[k]
    ctxw = jnp.dot(S, act, preferred_element_type=f32)
    wsum = jnp.sum(aw, axis=1, keepdims=True)
    ctx = jnp.dot(ctxw, atw_ref[...], preferred_element_type=f32) + wsum * atb_ref[...]

    mg = jnp.dot(ctx, gk_ref[...], preferred_element_type=f32) + gbi_ref[...]
    hg = jnp.dot(af, grk_ref[...], preferred_element_type=f32) + gbr_ref[...]
    z = jax.nn.sigmoid(mg[:, :FP] + hg[:, :FP])
    r = jax.nn.sigmoid(mg[:, FP:2 * FP] + hg[:, FP:2 * FP])
    hh = jnp.tanh(mg[:, 2 * FP:] + r * hg[:, 2 * FP:])
    h = z * af + (1.0 - z) * hh

    mol = jnp.sum(h * am, axis=0, keepdims=True)

    aft = jnp.dot(act, maw_ref[...], preferred_element_type=f32) + mab_ref[...]
    q = jnp.dot(act, molbw_ref[...], preferred_element_type=f32)
    mmask = jnp.where(am == 0.0, _NEG, 0.0)
    for _ in range(2):
        actm = _lk(mol)
        psc = jnp.dot(actm, mola_ref[...], preferred_element_type=f32)
        ms = _lk(psc + q + molb_ref[...]) + mmask
        mmax = jnp.max(ms, axis=0, keepdims=True)
        me = jnp.exp(ms - mmax)
        mw = me / jnp.sum(me, axis=0, keepdims=True) * am
        mcs = jnp.sum(mw * aft, axis=0, keepdims=True)
        mc = jnp.where(mcs > 0, mcs, jnp.exp(jnp.minimum(mcs, 0.0)) - 1.0)
        a1 = jnp.dot(mc, mgk_ref[...], preferred_element_type=f32) + mgbi_ref[...]
        a2 = jnp.dot(mol, mgrk_ref[...], preferred_element_type=f32) + mgbr_ref[...]
        z2 = jax.nn.sigmoid(a1[:, :FP] + a2[:, :FP])
        r2_ = jax.nn.sigmoid(a1[:, FP:2 * FP] + a2[:, FP:2 * FP])
        hh2 = jnp.tanh(a1[:, 2 * FP:] + r2_ * a2[:, 2 * FP:])
        mol = z2 * mol + (1.0 - z2) * hh2

    r1 = _lk(jnp.dot(mol, l1w_ref[...], preferred_element_type=f32) + l1b_ref[...])
    r2 = _lk(jnp.dot(r1, l2w_ref[...], preferred_element_type=f32) + l2b_ref[...])
    o = jnp.dot(r2, ow_ref[...], preferred_element_type=f32) + ob_ref[...]
    out_ref[0] = o


def kernel(atom_list, bond_list, atom_degree_list, bond_degree_list, atom_mask, params):
    p = params
    adl = atom_degree_list.astype(jnp.int32)
    am3 = atom_mask[..., None].astype(jnp.float32)
    alw = p['align_w_2']
    molw = p['mol_align_w']

    def r2(v):
        return v.reshape(1, -1).astype(jnp.float32)

    mol_spec = lambda shape: pl.BlockSpec(shape, lambda b: (b, 0, 0))
    par_spec = lambda shape: pl.BlockSpec(shape, lambda b: (0, 0))

    operands = [
        atom_list, adl, am3,
        p['atom_fc_w'], r2(p['atom_fc_b']),
        alw[:FP], alw[FP:], r2(p['align_b_2']),
        p['attend_w_2'], r2(p['attend_b_2']),
        p['gru_k_2'], p['gru_rk_2'], r2(p['gru_bi_2']), r2(p['gru_br_2']),
        molw[:FP], molw[FP:], r2(p['mol_align_b']),
        p['mol_attend_w'], r2(p['mol_attend_b']),
        p['mol_gru_k'], p['mol_gru_rk'], r2(p['mol_gru_bi']), r2(p['mol_gru_br']),
        p['lin1_w'], r2(p['lin1_b']), p['lin2_w'], r2(p['lin2_b']),
        p['out_w'], r2(p['out_b']),
    ]
    in_specs = [mol_spec((1, L, AF)), mol_spec((1, L, K)), mol_spec((1, L, 1))]
    in_specs += [par_spec(op.shape) for op in operands[3:]]

    out = pl.pallas_call(
        _fp_kernel,
        grid=(B,),
        in_specs=in_specs,
        out_specs=pl.BlockSpec((1, 1, 1), lambda b: (b, 0, 0)),
        out_shape=jax.ShapeDtypeStruct((B, 1, 1), jnp.float32),
    )(*operands)
    return out.reshape(B, 1)


# speedup vs baseline: 13.4967x; 1.4815x over previous
import jax
import jax.numpy as jnp
from jax.experimental import pallas as pl

FP = 64
B = 256
L = 256
K = 6
AF = 39
MP = 4

_NEG = -9e8


def _lk(x):
    return jax.nn.leaky_relu(x, 0.2)


def _fp_kernel(x_ref, idx_ref, am_ref,
               wa_ref, ba_ref,
               alwa_ref, alwb_ref, alb_ref,
               atw_ref, atb_ref,
               gk_ref, grk_ref, gbi_ref, gbr_ref,
               mola_ref, molbw_ref, molb_ref,
               maw_ref, mab_ref,
               mgk_ref, mgrk_ref, mgbi_ref, mgbr_ref,
               l1w_ref, l1b_ref, l2w_ref, l2b_ref,
               ow_ref, ob_ref,
               out_ref):
    f32 = jnp.float32
    iota = jax.lax.broadcasted_iota(jnp.int32, (L, L), 1)

    def atom_stage(m):
        x = x_ref[m]
        idx = idx_ref[m]
        am = am_ref[m]

        af = _lk(jnp.dot(x, wa_ref[...], preferred_element_type=f32) + ba_ref[...])
        act = _lk(af)

        g = jnp.dot(act, alwb_ref[...], preferred_element_type=f32)
        a_sc = jnp.dot(af, alwa_ref[...], preferred_element_type=f32)
        ohs = []
        gvs = []
        for k in range(K):
            oh = (idx[:, k:k + 1] == iota).astype(f32)
            ohs.append(oh)
            gvs.append(jnp.dot(oh, g, preferred_element_type=f32))
        gv = jnp.concatenate(gvs, axis=1)
        pad = idx == (L - 1)
        sc = _lk(a_sc + gv + alb_ref[...]) + jnp.where(pad, _NEG, 0.0)
        mx = jnp.max(sc, axis=1, keepdims=True)
        e = jnp.exp(sc - mx)
        aw = e / jnp.sum(e, axis=1, keepdims=True)
        aw = aw * jnp.where(pad, 0.0, 1.0)

        S = aw[:, 0:1] * ohs[0]
        for k in range(1, K):
            S = S + aw[:, k:k + 1] * ohs[k]
        ctxw = jnp.dot(S, act, preferred_element_type=f32)
        wsum = jnp.sum(aw, axis=1, keepdims=True)
        ctx = jnp.dot(ctxw, atw_ref[...], preferred_element_type=f32) + wsum * atb_ref[...]

        mg = jnp.dot(ctx, gk_ref[...], preferred_element_type=f32) + gbi_ref[...]
        hg = jnp.dot(af, grk_ref[...], preferred_element_type=f32) + gbr_ref[...]
        z = jax.nn.sigmoid(mg[:, :FP] + hg[:, :FP])
        r = jax.nn.sigmoid(mg[:, FP:2 * FP] + hg[:, FP:2 * FP])
        hh = jnp.tanh(mg[:, 2 * FP:] + r * hg[:, 2 * FP:])
        h = z * af + (1.0 - z) * hh

        mol_m = jnp.sum(h * am, axis=0, keepdims=True)
        aft = jnp.dot(act, maw_ref[...], preferred_element_type=f32) + mab_ref[...]
        q = jnp.dot(act, molbw_ref[...], preferred_element_type=f32)
        mmask = jnp.where(am == 0.0, _NEG, 0.0)
        return mol_m, aft, q, mmask, am

    per_mol = [atom_stage(m) for m in range(MP)]
    mol = jnp.concatenate([pm[0] for pm in per_mol], axis=0)

    for _ in range(2):
        actm = _lk(mol)
        psc = jnp.dot(actm, mola_ref[...], preferred_element_type=f32)
        mcs = []
        for m in range(MP):
            _, aft, q, mmask, am = per_mol[m]
            ms = _lk(psc[m:m + 1, 0:1] + q + molb_ref[...]) + mmask
            mmax = jnp.max(ms, axis=0, keepdims=True)
            me = jnp.exp(ms - mmax)
            mw = me / jnp.sum(me, axis=0, keepdims=True) * am
            mcs.append(jnp.sum(mw * aft, axis=0, keepdims=True))
        mcs_c = jnp.concatenate(mcs, axis=0)
        mc = jnp.where(mcs_c > 0, mcs_c, jnp.exp(jnp.minimum(mcs_c, 0.0)) - 1.0)
        a1 = jnp.dot(mc, mgk_ref[...], preferred_element_type=f32) + mgbi_ref[...]
        a2 = jnp.dot(mol, mgrk_ref[...], preferred_element_type=f32) + mgbr_ref[...]
        z2 = jax.nn.sigmoid(a1[:, :FP] + a2[:, :FP])
        r2_ = jax.nn.sigmoid(a1[:, FP:2 * FP] + a2[:, FP:2 * FP])
        hh2 = jnp.tanh(a1[:, 2 * FP:] + r2_ * a2[:, 2 * FP:])
        mol = z2 * mol + (1.0 - z2) * hh2

    r1 = _lk(jnp.dot(mol, l1w_ref[...], preferred_element_type=f32) + l1b_ref[...])
    r2 = _lk(jnp.dot(r1, l2w_ref[...], preferred_element_type=f32) + l2b_ref[...])
    o = jnp.dot(r2, ow_ref[...], preferred_element_type=f32) + ob_ref[...]
    out_ref[...] = o[:, :, None]


def kernel(atom_list, bond_list, atom_degree_list, bond_degree_list, atom_mask, params):
    p = params
    adl = atom_degree_list.astype(jnp.int32)
    am3 = atom_mask[..., None].astype(jnp.float32)
    alw = p['align_w_2']
    molw = p['mol_align_w']

    def r2(v):
        return v.reshape(1, -1).astype(jnp.float32)

    mol_spec = lambda shape: pl.BlockSpec(shape, lambda b: (b, 0, 0))
    par_spec = lambda shape: pl.BlockSpec(shape, lambda b: (0, 0))

    operands = [
        atom_list, adl, am3,
        p['atom_fc_w'], r2(p['atom_fc_b']),
        alw[:FP], alw[FP:], r2(p['align_b_2']),
        p['attend_w_2'], r2(p['attend_b_2']),
        p['gru_k_2'], p['gru_rk_2'], r2(p['gru_bi_2']), r2(p['gru_br_2']),
        molw[:FP], molw[FP:], r2(p['mol_align_b']),
        p['mol_attend_w'], r2(p['mol_attend_b']),
        p['mol_gru_k'], p['mol_gru_rk'], r2(p['mol_gru_bi']), r2(p['mol_gru_br']),
        p['lin1_w'], r2(p['lin1_b']), p['lin2_w'], r2(p['lin2_b']),
        p['out_w'], r2(p['out_b']),
    ]
    in_specs = [mol_spec((MP, L, AF)), mol_spec((MP, L, K)), mol_spec((MP, L, 1))]
    in_specs += [par_spec(op.shape) for op in operands[3:]]

    out = pl.pallas_call(
        _fp_kernel,
        grid=(B // MP,),
        in_specs=in_specs,
        out_specs=pl.BlockSpec((MP, 1, 1), lambda b: (b, 0, 0)),
        out_shape=jax.ShapeDtypeStruct((B, 1, 1), jnp.float32),
    )(*operands)
    return out.reshape(B, 1)


# speedup vs baseline: 14.4066x; 1.0674x over previous
import jax
import jax.numpy as jnp
from jax.experimental import pallas as pl

FP = 64
B = 256
L = 256
K = 6
AF = 39
MP = 8

_NEG = -9e8


def _lk(x):
    return jax.nn.leaky_relu(x, 0.2)


def _fp_kernel(x_ref, idx_ref, am_ref,
               wa_ref, ba_ref,
               alwa_ref, alwb_ref, alb_ref,
               atw_ref, atb_ref,
               gk_ref, grk_ref, gbi_ref, gbr_ref,
               mola_ref, molbw_ref, molb_ref,
               maw_ref, mab_ref,
               mgk_ref, mgrk_ref, mgbi_ref, mgbr_ref,
               l1w_ref, l1b_ref, l2w_ref, l2b_ref,
               ow_ref, ob_ref,
               out_ref):
    f32 = jnp.float32
    iota = jax.lax.broadcasted_iota(jnp.int32, (L, L), 1)

    def atom_stage(m):
        x = x_ref[m]
        idx = idx_ref[m]
        am = am_ref[m]

        af = _lk(jnp.dot(x, wa_ref[...], preferred_element_type=f32) + ba_ref[...])
        act = _lk(af)

        g = jnp.dot(act, alwb_ref[...], preferred_element_type=f32)
        a_sc = jnp.dot(af, alwa_ref[...], preferred_element_type=f32)
        gvs = []
        for k in range(K):
            oh = (idx[:, k:k + 1] == iota).astype(f32)
            gvs.append(jnp.dot(oh, g, preferred_element_type=f32))
        gv = jnp.concatenate(gvs, axis=1)
        pad = idx == (L - 1)
        sc = _lk(a_sc + gv + alb_ref[...]) + jnp.where(pad, _NEG, 0.0)
        mx = jnp.max(sc, axis=1, keepdims=True)
        e = jnp.exp(sc - mx)
        aw = e / jnp.sum(e, axis=1, keepdims=True)
        aw = aw * jnp.where(pad, 0.0, 1.0)

        S = jnp.where(idx[:, 0:1] == iota, aw[:, 0:1], 0.0)
        for k in range(1, K):
            S = S + jnp.where(idx[:, k:k + 1] == iota, aw[:, k:k + 1], 0.0)
        ctxw = jnp.dot(S, act, preferred_element_type=f32)
        wsum = jnp.sum(aw, axis=1, keepdims=True)
        ctx = jnp.dot(ctxw, atw_ref[...], preferred_element_type=f32) + wsum * atb_ref[...]

        mg = jnp.dot(ctx, gk_ref[...], preferred_element_type=f32) + gbi_ref[...]
        hg = jnp.dot(af, grk_ref[...], preferred_element_type=f32) + gbr_ref[...]
        z = jax.nn.sigmoid(mg[:, :FP] + hg[:, :FP])
        r = jax.nn.sigmoid(mg[:, FP:2 * FP] + hg[:, FP:2 * FP])
        hh = jnp.tanh(mg[:, 2 * FP:] + r * hg[:, 2 * FP:])
        h = z * af + (1.0 - z) * hh

        mol_m = jnp.sum(h * am, axis=0, keepdims=True)
        aft = jnp.dot(act, maw_ref[...], preferred_element_type=f32) + mab_ref[...]
        q = jnp.dot(act, molbw_ref[...], preferred_element_type=f32)
        mmask = jnp.where(am == 0.0, _NEG, 0.0)
        return mol_m, aft, q, mmask, am

    per_mol = [atom_stage(m) for m in range(MP)]
    mol = jnp.concatenate([pm[0] for pm in per_mol], axis=0)

    for _ in range(2):
        actm = _lk(mol)
        psc = jnp.dot(actm, mola_ref[...], preferred_element_type=f32)
        mcs = []
        for m in range(MP):
            _, aft, q, mmask, am = per_mol[m]
            ms = _lk(psc[m:m + 1, 0:1] + q + molb_ref[...]) + mmask
            mmax = jnp.max(ms, axis=0, keepdims=True)
            me = jnp.exp(ms - mmax)
            mw = me / jnp.sum(me, axis=0, keepdims=True) * am
            mcs.append(jnp.sum(mw * aft, axis=0, keepdims=True))
        mcs_c = jnp.concatenate(mcs, axis=0)
        mc = jnp.where(mcs_c > 0, mcs_c, jnp.exp(jnp.minimum(mcs_c, 0.0)) - 1.0)
        a1 = jnp.dot(mc, mgk_ref[...], preferred_element_type=f32) + mgbi_ref[...]
        a2 = jnp.dot(mol, mgrk_ref[...], preferred_element_type=f32) + mgbr_ref[...]
        z2 = jax.nn.sigmoid(a1[:, :FP] + a2[:, :FP])
        r2_ = jax.nn.sigmoid(a1[:, FP:2 * FP] + a2[:, FP:2 * FP])
        hh2 = jnp.tanh(a1[:, 2 * FP:] + r2_ * a2[:, 2 * FP:])
        mol = z2 * mol + (1.0 - z2) * hh2

    r1 = _lk(jnp.dot(mol, l1w_ref[...], preferred_element_type=f32) + l1b_ref[...])
    r2 = _lk(jnp.dot(r1, l2w_ref[...], preferred_element_type=f32) + l2b_ref[...])
    o = jnp.dot(r2, ow_ref[...], preferred_element_type=f32) + ob_ref[...]
    out_ref[...] = o[:, :, None]


def kernel(atom_list, bond_list, atom_degree_list, bond_degree_list, atom_mask, params):
    p = params
    adl = atom_degree_list.astype(jnp.int32)
    am3 = atom_mask[..., None].astype(jnp.float32)
    alw = p['align_w_2']
    molw = p['mol_align_w']

    def r2(v):
        return v.reshape(1, -1).astype(jnp.float32)

    mol_spec = lambda shape: pl.BlockSpec(shape, lambda b: (b, 0, 0))
    par_spec = lambda shape: pl.BlockSpec(shape, lambda b: (0, 0))

    operands = [
        atom_list, adl, am3,
        p['atom_fc_w'], r2(p['atom_fc_b']),
        alw[:FP], alw[FP:], r2(p['align_b_2']),
        p['attend_w_2'], r2(p['attend_b_2']),
        p['gru_k_2'], p['gru_rk_2'], r2(p['gru_bi_2']), r2(p['gru_br_2']),
        molw[:FP], molw[FP:], r2(p['mol_align_b']),
        p['mol_attend_w'], r2(p['mol_attend_b']),
        p['mol_gru_k'], p['mol_gru_rk'], r2(p['mol_gru_bi']), r2(p['mol_gru_br']),
        p['lin1_w'], r2(p['lin1_b']), p['lin2_w'], r2(p['lin2_b']),
        p['out_w'], r2(p['out_b']),
    ]
    in_specs = [mol_spec((MP, L, AF)), mol_spec((MP, L, K)), mol_spec((MP, L, 1))]
    in_specs += [par_spec(op.shape) for op in operands[3:]]

    out = pl.pallas_call(
        _fp_kernel,
        grid=(B // MP,),
        in_specs=in_specs,
        out_specs=pl.BlockSpec((MP, 1, 1), lambda b: (b, 0, 0)),
        out_shape=jax.ShapeDtypeStruct((B, 1, 1), jnp.float32),
    )(*operands)
    return out.reshape(B, 1)
